# baseline (speedup 1.0000x reference)
import functools

import jax
import jax.numpy as jnp
from jax import lax
from jax.experimental import pallas as pl
from jax.experimental.pallas import tpu as pltpu
import jax.experimental.pallas.tpu_sc as plsc

E = 320000
N = 10000
D = 128
DE = 16
HIDDEN = 128

NC = 2
NS = 16
CHUNK = 128
CHUNKS_PER_W = 20
CHUNKS_PER_W_PAD = 24
EDGES_PER_W = CHUNK * CHUNKS_PER_W
SLICE = NC * NS * EDGES_PER_W
NSLICES = 4
E_PAD = NSLICES * SLICE
ROWS_PER_TILE = 632
N_PAD = NS * ROWS_PER_TILE

BLK = 2560
BLKS_PER_SLICE = SLICE // BLK


def _mlp_body(msg_ref, xi_ref, xj_ref, e_ref, w1a_ref, w1b_ref, w1c_ref,
              b1_ref, w2_ref, b2_ref, out_ref):
    h = jnp.dot(xj_ref[...], w1a_ref[...], preferred_element_type=jnp.float32)
    h = h + jnp.dot(e_ref[...], w1b_ref[...], preferred_element_type=jnp.float32)
    h = h + jnp.dot(xi_ref[...], w1c_ref[...], preferred_element_type=jnp.float32)
    h = h + b1_ref[...]
    h = jnp.where(h >= 0, h, 0.01 * h)
    w = jnp.sum(h * w2_ref[...], axis=1, keepdims=True) + b2_ref[0, 0]
    w = jnp.tanh(w)
    out_ref[...] = w * msg_ref[...]


def _gated_mlp_slice(s, msg, x_i, x_j, e_ij, w1a, w1b, w1c, b1r, w2r, b2r):
    off = s * BLKS_PER_SLICE
    nblk = min(E - s * SLICE, SLICE) // BLK
    edge = lambda i: (i + off, 0)
    const = lambda i: (0, 0)
    return pl.pallas_call(
        _mlp_body,
        grid=(nblk,),
        in_specs=[
            pl.BlockSpec((BLK, D), edge),
            pl.BlockSpec((BLK, D), edge),
            pl.BlockSpec((BLK, D), edge),
            pl.BlockSpec((BLK, DE), edge),
            pl.BlockSpec((D, HIDDEN), const),
            pl.BlockSpec((DE, HIDDEN), const),
            pl.BlockSpec((D, HIDDEN), const),
            pl.BlockSpec((1, HIDDEN), const),
            pl.BlockSpec((1, HIDDEN), const),
            pl.BlockSpec((1, 1), const),
        ],
        out_specs=pl.BlockSpec((BLK, D), lambda i: (i, 0)),
        out_shape=jax.ShapeDtypeStruct((SLICE, D), jnp.float32),
    )(msg, x_i, x_j, e_ij, w1a, w1b, w1c, b1r, w2r, b2r)


NPAIRS = CHUNKS_PER_W


def _sc_scatter_body(gated_hbm, idx_hbm, zrows_hbm, out_hbm,
                     idxbuf, b0, b1, s0, s1, accum):
    c = lax.axis_index("c")
    s = lax.axis_index("s")
    w = c * NS + s
    pltpu.sync_copy(idx_hbm.at[w], idxbuf)
    pltpu.sync_copy(zrows_hbm, accum.at[pl.ds(s * ROWS_PER_TILE, ROWS_PER_TILE)])
    plsc.subcore_barrier()

    row0 = w * EDGES_PER_W

    def start(pair, buf, sem):
        pltpu.async_copy(gated_hbm.at[pl.ds(row0 + pair * CHUNK, CHUNK)], buf, sem)

    def wait(pair, buf, sem):
        pltpu.make_async_copy(
            gated_hbm.at[pl.ds(row0 + pair * CHUNK, CHUNK)], buf, sem).wait()

    def scatter(pair, buf):
        pltpu.sync_copy(buf, accum.at[idxbuf.at[pair]], add=True)

    start(0, b0, s0)

    def body(q, carry):
        p0 = 2 * q
        start(p0 + 1, b1, s1)
        wait(p0, b0, s0)
        scatter(p0, b0)
        start(p0 + 2, b0, s0)
        wait(p0 + 1, b1, s1)
        scatter(p0 + 1, b1)
        return carry

    lax.fori_loop(0, NPAIRS // 2 - 1, body, 0)
    start(NPAIRS - 1, b1, s1)
    wait(NPAIRS - 2, b0, s0)
    scatter(NPAIRS - 2, b0)
    wait(NPAIRS - 1, b1, s1)
    scatter(NPAIRS - 1, b1)

    plsc.subcore_barrier()
    pltpu.sync_copy(accum.at[pl.ds(s * ROWS_PER_TILE, ROWS_PER_TILE)],
                    out_hbm.at[c, pl.ds(s * ROWS_PER_TILE, ROWS_PER_TILE)])


_sc_scatter = functools.partial(
    pl.kernel,
    out_type=jax.ShapeDtypeStruct((NC, N_PAD, D), jnp.float32),
    mesh=plsc.VectorSubcoreMesh(core_axis_name="c", subcore_axis_name="s"),
    scratch_types=[
        pltpu.VMEM((CHUNKS_PER_W_PAD, CHUNK), jnp.int32),
        pltpu.VMEM((CHUNK, D), jnp.float32),
        pltpu.VMEM((CHUNK, D), jnp.float32),
        pltpu.SemaphoreType.DMA,
        pltpu.SemaphoreType.DMA,
        pltpu.VMEM_SHARED((N_PAD, D), jnp.float32),
    ],
)(_sc_scatter_body)


CBLK = 2000


def _combine_body(*refs):
    out_ref = refs[-1]
    acc = refs[0][...]
    for r in refs[1:-1]:
        acc = acc + r[...]
    out_ref[...] = acc


def _combine(partials):
    specs = []
    args = []
    for p in partials:
        for core in (0, 1):
            specs.append(
                pl.BlockSpec((None, CBLK, D), functools.partial(
                    lambda i, c: (c, i, 0), c=core)))
            args.append(p)
    return pl.pallas_call(
        _combine_body,
        grid=(N // CBLK,),
        in_specs=specs,
        out_specs=pl.BlockSpec((CBLK, D), lambda i: (i, 0)),
        out_shape=jax.ShapeDtypeStruct((N, D), jnp.float32),
    )(*args)


def kernel(msg, x_i, x_j, e_ij, index, num_nodes, W1, b1, W2, b2):
    w1a = W1[:D]
    w1b = W1[D:D + DE]
    w1c = W1[D + DE:]
    b1r = b1.reshape(1, HIDDEN)
    w2r = W2.reshape(1, HIDDEN)
    b2r = b2.reshape(1, 1)

    idx = index.astype(jnp.int32)
    idx_pad = jnp.concatenate(
        [idx, jnp.full((E_PAD - E,), N, jnp.int32)])
    idx4 = jnp.pad(
        idx_pad.reshape(NSLICES, NC * NS, CHUNKS_PER_W, CHUNK),
        ((0, 0), (0, 0), (0, CHUNKS_PER_W_PAD - CHUNKS_PER_W), (0, 0)),
        constant_values=N)
    zrows = jnp.zeros((ROWS_PER_TILE, D), jnp.float32)

    partials = []
    for s in range(NSLICES):
        gated = _gated_mlp_slice(s, msg, x_i, x_j, e_ij,
                                 w1a, w1b, w1c, b1r, w2r, b2r)
        partials.append(_sc_scatter(gated, idx4[s], zrows))
    return _combine(partials)
